# Initial kernel scaffold; baseline (speedup 1.0000x reference)
import jax
import jax.numpy as jnp
from jax.experimental import pallas as pl


def kernel(full_imgs, center, bbox_size):
    raise NotImplementedError("write your pallas kernel here")


# speedup vs baseline: 1.2808x; 1.2808x over previous
import functools

import jax
import jax.numpy as jnp
from jax import lax
from jax.experimental import pallas as pl
from jax.experimental.pallas import tpu as pltpu
from jax.experimental.pallas import tpu_sc as plsc

CROP = 256
B, C, H, W = 4, 3, 2048, 2048
NROWS = B * C * CROP
NSUB = 32
ROWS_PER_SUB = NROWS // NSUB
GROUP = 8
NGROUPS = ROWS_PER_SUB // GROUP
NBLK = 8


def _sc_bilinear(img_blocks, idx_all, xloc, wx1, wy, boff):
    mesh = plsc.VectorSubcoreMesh(core_axis_name="c", subcore_axis_name="s",
                                  num_cores=2, num_subcores=16)

    @functools.partial(
        pl.kernel,
        out_type=jax.ShapeDtypeStruct((NROWS * CROP,), jnp.float32),
        mesh=mesh,
        compiler_params=pltpu.CompilerParams(needs_layout_passes=False),
        scratch_types=[
            pltpu.VMEM((2 * GROUP * NBLK,), jnp.int32),
            pltpu.VMEM((B * CROP,), jnp.int32),
            pltpu.VMEM((B * CROP,), jnp.float32),
            pltpu.VMEM((ROWS_PER_SUB,), jnp.float32),
            pltpu.VMEM((ROWS_PER_SUB,), jnp.int32),
            pltpu.VMEM((2 * GROUP * NBLK, 128), jnp.float32),
            pltpu.VMEM((GROUP * CROP,), jnp.float32),
            pltpu.SemaphoreType.DMA,
        ],
    )
    def k(img_hbm, idx_hbm, xloc_hbm, wx1_hbm, wy_hbm, boff_hbm, out_hbm,
          idx_v, xloc_v, wx1_v, wy_v, boff_v, buf_v, outb_v, sem):
        wid = lax.axis_index("s") * 2 + lax.axis_index("c")
        rbase = wid * ROWS_PER_SUB
        pltpu.sync_copy(xloc_hbm, xloc_v)
        pltpu.sync_copy(wx1_hbm, wx1_v)
        pltpu.sync_copy(wy_hbm.at[pl.ds(rbase, ROWS_PER_SUB)], wy_v)
        pltpu.sync_copy(boff_hbm.at[pl.ds(rbase, ROWS_PER_SUB)], boff_v)

        def group_body(g, carry):
            gsize = 2 * GROUP * NBLK
            pltpu.sync_copy(idx_hbm.at[pl.ds((rbase + g * GROUP) * 16, gsize)],
                            idx_v)
            pltpu.async_copy(img_hbm.at[idx_v], buf_v, sem).wait()

            for t in range(GROUP):
                ridx = jnp.full((16,), g * GROUP + t, dtype=jnp.int32)
                wyv = plsc.load_gather(wy_v, [ridx])
                bofv = plsc.load_gather(boff_v, [ridx])

                def chunk_body(kk, carry2):
                    qi = kk * 16 + lax.iota(jnp.int32, 16)
                    xq = bofv + qi
                    xl = plsc.load_gather(xloc_v, [xq])
                    w1 = plsc.load_gather(wx1_v, [xq])
                    r0 = t * 2 * NBLK + lax.shift_right_logical(xl, 7)
                    c0 = lax.bitwise_and(xl, 127)
                    xl1 = xl + 1
                    r1 = t * 2 * NBLK + lax.shift_right_logical(xl1, 7)
                    c1 = lax.bitwise_and(xl1, 127)
                    a0 = plsc.load_gather(buf_v, [r0, c0])
                    a1 = plsc.load_gather(buf_v, [r1, c1])
                    b0 = plsc.load_gather(buf_v, [r0 + NBLK, c0])
                    b1 = plsc.load_gather(buf_v, [r1 + NBLK, c1])
                    top = a0 + w1 * (a1 - a0)
                    bot = b0 + w1 * (b1 - b0)
                    val = top + wyv * (bot - top)
                    outb_v[pl.ds(t * CROP + kk * 16, 16)] = val
                    return carry2

                lax.fori_loop(0, CROP // 16, chunk_body, 0)

            pltpu.sync_copy(outb_v,
                            out_hbm.at[pl.ds((rbase + g * GROUP) * CROP,
                                             GROUP * CROP)])
            return carry

        lax.fori_loop(0, NGROUPS, group_body, 0)

    return k(img_blocks, idx_all, xloc, wx1, wy, boff)


def kernel(full_imgs, center, bbox_size):
    crop = CROP
    zeros = jnp.zeros((B,), dtype=full_imgs.dtype)
    ones = jnp.ones((B,), dtype=full_imgs.dtype)
    s = bbox_size
    cx = center[:, 0]
    cy = center[:, 1]
    transforms = jnp.stack([
        jnp.stack([s, zeros, cx - s * 0.5], axis=1),
        jnp.stack([zeros, s, cy - s * 0.5], axis=1),
        jnp.stack([zeros, zeros, ones], axis=1)], axis=1)
    a = 2.0 * (crop - 1) / s
    hd_to_crop = jnp.stack([
        jnp.stack([a, zeros, -(cx - s * 0.5) * a - 1.0], axis=1),
        jnp.stack([zeros, a, -(cy - s * 0.5) * a - 1.0], axis=1),
        jnp.stack([zeros, zeros, ones], axis=1)], axis=1)

    sx = 2.0 / (W - 1) * ones
    sy = 2.0 / (H - 1) * ones
    size_bbox_sizer = jnp.stack([
        jnp.stack([sx, zeros, -ones], axis=1),
        jnp.stack([zeros, sy, -ones], axis=1),
        jnp.stack([zeros, zeros, ones], axis=1)], axis=1)
    full_transform = jnp.einsum('bij,bjk->bik', size_bbox_sizer, transforms)
    x1d = jnp.arange(crop, dtype=jnp.float32) / (crop - 1)
    gy, gx = jnp.meshgrid(x1d, x1d, indexing='ij')
    points = jnp.stack([gy.reshape(-1), gx.reshape(-1)], axis=1)
    batch_grid = jnp.broadcast_to(points[None], (B, crop * crop, 2))
    sg = (jnp.einsum('bij,bnj->bni', full_transform[:, :2, :2], batch_grid)
          + full_transform[:, :2, 2][:, None, :])
    sampling_grid = jnp.swapaxes(sg.reshape(B, crop, crop, 2), 1, 2)

    sg_x = sampling_grid[:, 0, :, 0]
    sg_y = sampling_grid[:, :, 0, 1]
    ix = (sg_x + 1.0) * (W - 1) / 2.0
    iy = (sg_y + 1.0) * (H - 1) / 2.0
    x0f = jnp.floor(ix)
    y0f = jnp.floor(iy)
    x0 = x0f.astype(jnp.int32)
    y0 = y0f.astype(jnp.int32)
    wx1 = ix - x0f
    wy1 = iy - y0f

    xb0 = lax.shift_right_logical(x0[:, 0], 7)
    xloc = x0 - lax.shift_left(xb0, 7)[:, None]

    bc = (jnp.arange(B, dtype=jnp.int32)[:, None] * C
          + jnp.arange(C, dtype=jnp.int32)[None, :])
    blkbase = ((bc[:, :, None] * H + y0[:, None, :]) * 16
               + xb0[:, None, None])
    koff = jnp.arange(2 * NBLK, dtype=jnp.int32)
    koff = (koff % NBLK) + (koff // NBLK) * 16
    idx_all = (blkbase[:, :, :, None] + koff[None, None, None, :]).reshape(-1)

    wy_flat = jnp.broadcast_to(wy1[:, None, :], (B, C, crop)).reshape(-1)
    boff = jnp.broadcast_to(
        (jnp.arange(B, dtype=jnp.int32) * crop)[:, None, None],
        (B, C, crop)).reshape(-1)

    img_blocks = full_imgs.reshape(B * C * H * 16, 128)
    out_flat = _sc_bilinear(img_blocks, idx_all, xloc.reshape(-1),
                            wx1.reshape(-1), wy_flat, boff)
    out_images = out_flat.reshape(B, C, crop, crop)

    return (out_images, sampling_grid.reshape(B, -1, 2), transforms,
            hd_to_crop)


# speedup vs baseline: 3.6707x; 2.8659x over previous
import functools

import jax
import jax.numpy as jnp
from jax import lax
from jax.experimental import pallas as pl
from jax.experimental.pallas import tpu as pltpu
from jax.experimental.pallas import tpu_sc as plsc

CROP = 256
B, C, H, W = 4, 3, 2048, 2048
NROWS = B * C * CROP
NSUB = 32
ROWS_PER_SUB = NROWS // NSUB
GROUP = 8
NGROUPS = ROWS_PER_SUB // GROUP


def _sc_bilinear(img_rows, idx_all, x0tab, wx1, wy, boff):
    mesh = plsc.VectorSubcoreMesh(core_axis_name="c", subcore_axis_name="s",
                                  num_cores=2, num_subcores=16)

    @functools.partial(
        pl.kernel,
        out_type=jax.ShapeDtypeStruct((NROWS * CROP,), jnp.float32),
        mesh=mesh,
        compiler_params=pltpu.CompilerParams(needs_layout_passes=False),
        scratch_types=[
            pltpu.VMEM((2 * GROUP,), jnp.int32),
            pltpu.VMEM((B * CROP,), jnp.int32),
            pltpu.VMEM((B * CROP,), jnp.float32),
            pltpu.VMEM((ROWS_PER_SUB,), jnp.float32),
            pltpu.VMEM((ROWS_PER_SUB,), jnp.int32),
            pltpu.VMEM((2 * GROUP, W), jnp.float32),
            pltpu.VMEM((GROUP * CROP,), jnp.float32),
            pltpu.SemaphoreType.DMA,
        ],
    )
    def k(img_hbm, idx_hbm, x0_hbm, wx1_hbm, wy_hbm, boff_hbm, out_hbm,
          idx_v, x0_v, wx1_v, wy_v, boff_v, buf_v, outb_v, sem):
        wid = lax.axis_index("s") * 2 + lax.axis_index("c")
        rbase = wid * ROWS_PER_SUB
        pltpu.sync_copy(x0_hbm, x0_v)
        pltpu.sync_copy(wx1_hbm, wx1_v)
        pltpu.sync_copy(wy_hbm.at[pl.ds(rbase, ROWS_PER_SUB)], wy_v)
        pltpu.sync_copy(boff_hbm.at[pl.ds(rbase, ROWS_PER_SUB)], boff_v)

        def group_body(g, carry):
            pltpu.sync_copy(idx_hbm.at[pl.ds((rbase + g * GROUP) * 2,
                                             2 * GROUP)], idx_v)
            pltpu.async_copy(img_hbm.at[idx_v], buf_v, sem).wait()

            for t in range(GROUP):
                ridx = jnp.full((16,), g * GROUP + t, dtype=jnp.int32)
                wyv = plsc.load_gather(wy_v, [ridx])
                bofv = plsc.load_gather(boff_v, [ridx])
                ra = jnp.full((16,), 2 * t, dtype=jnp.int32)
                rb = jnp.full((16,), 2 * t + 1, dtype=jnp.int32)

                def chunk_body(kk, carry2):
                    qi = kk * 16 + lax.iota(jnp.int32, 16)
                    xq = bofv + qi
                    xi = plsc.load_gather(x0_v, [xq])
                    w1 = plsc.load_gather(wx1_v, [xq])
                    xi1 = xi + 1
                    a0 = plsc.load_gather(buf_v, [ra, xi])
                    a1 = plsc.load_gather(buf_v, [ra, xi1])
                    b0 = plsc.load_gather(buf_v, [rb, xi])
                    b1 = plsc.load_gather(buf_v, [rb, xi1])
                    top = a0 + w1 * (a1 - a0)
                    bot = b0 + w1 * (b1 - b0)
                    val = top + wyv * (bot - top)
                    outb_v[pl.ds(t * CROP + kk * 16, 16)] = val
                    return carry2

                lax.fori_loop(0, CROP // 16, chunk_body, 0)

            pltpu.sync_copy(outb_v,
                            out_hbm.at[pl.ds((rbase + g * GROUP) * CROP,
                                             GROUP * CROP)])
            return carry

        lax.fori_loop(0, NGROUPS, group_body, 0)

    return k(img_rows, idx_all, x0tab, wx1, wy, boff)


def kernel(full_imgs, center, bbox_size):
    crop = CROP
    zeros = jnp.zeros((B,), dtype=full_imgs.dtype)
    ones = jnp.ones((B,), dtype=full_imgs.dtype)
    s = bbox_size
    cx = center[:, 0]
    cy = center[:, 1]
    transforms = jnp.stack([
        jnp.stack([s, zeros, cx - s * 0.5], axis=1),
        jnp.stack([zeros, s, cy - s * 0.5], axis=1),
        jnp.stack([zeros, zeros, ones], axis=1)], axis=1)
    a = 2.0 * (crop - 1) / s
    hd_to_crop = jnp.stack([
        jnp.stack([a, zeros, -(cx - s * 0.5) * a - 1.0], axis=1),
        jnp.stack([zeros, a, -(cy - s * 0.5) * a - 1.0], axis=1),
        jnp.stack([zeros, zeros, ones], axis=1)], axis=1)

    sx = 2.0 / (W - 1) * ones
    sy = 2.0 / (H - 1) * ones
    size_bbox_sizer = jnp.stack([
        jnp.stack([sx, zeros, -ones], axis=1),
        jnp.stack([zeros, sy, -ones], axis=1),
        jnp.stack([zeros, zeros, ones], axis=1)], axis=1)
    full_transform = jnp.einsum('bij,bjk->bik', size_bbox_sizer, transforms)
    x1d = jnp.arange(crop, dtype=jnp.float32) / (crop - 1)
    gy, gx = jnp.meshgrid(x1d, x1d, indexing='ij')
    points = jnp.stack([gy.reshape(-1), gx.reshape(-1)], axis=1)
    batch_grid = jnp.broadcast_to(points[None], (B, crop * crop, 2))
    sg = (jnp.einsum('bij,bnj->bni', full_transform[:, :2, :2], batch_grid)
          + full_transform[:, :2, 2][:, None, :])
    sampling_grid = jnp.swapaxes(sg.reshape(B, crop, crop, 2), 1, 2)

    sg_x = sampling_grid[:, 0, :, 0]
    sg_y = sampling_grid[:, :, 0, 1]
    ix = (sg_x + 1.0) * (W - 1) / 2.0
    iy = (sg_y + 1.0) * (H - 1) / 2.0
    x0f = jnp.floor(ix)
    y0f = jnp.floor(iy)
    x0 = x0f.astype(jnp.int32)
    y0 = y0f.astype(jnp.int32)
    wx1 = ix - x0f
    wy1 = iy - y0f

    bc = (jnp.arange(B, dtype=jnp.int32)[:, None] * C
          + jnp.arange(C, dtype=jnp.int32)[None, :])
    rowid = bc[:, :, None] * H + y0[:, None, :]
    idx_all = jnp.stack([rowid, rowid + 1], axis=-1).reshape(-1)

    wy_flat = jnp.broadcast_to(wy1[:, None, :], (B, C, crop)).reshape(-1)
    boff = jnp.broadcast_to(
        (jnp.arange(B, dtype=jnp.int32) * crop)[:, None, None],
        (B, C, crop)).reshape(-1)

    img_rows = full_imgs.reshape(B * C * H, W)
    out_flat = _sc_bilinear(img_rows, idx_all, x0.reshape(-1),
                            wx1.reshape(-1), wy_flat, boff)
    out_images = out_flat.reshape(B, C, crop, crop)

    return (out_images, sampling_grid.reshape(B, -1, 2), transforms,
            hd_to_crop)


# speedup vs baseline: 4.5049x; 1.2273x over previous
import functools

import jax
import jax.numpy as jnp
from jax import lax
from jax.experimental import pallas as pl
from jax.experimental.pallas import tpu as pltpu
from jax.experimental.pallas import tpu_sc as plsc

CROP = 256
B, C, H, W = 4, 3, 2048, 2048
NROWS = B * C * CROP
NSUB = 32
ROWS_PER_SUB = NROWS // NSUB
GROUP = 8
NGROUPS = ROWS_PER_SUB // GROUP


WIN = 1024


def _sc_bilinear(img_rows, idx_all, xloc, wx1, wy, c0tab):
    mesh = plsc.VectorSubcoreMesh(core_axis_name="c", subcore_axis_name="s",
                                  num_cores=2, num_subcores=16)

    @functools.partial(
        pl.kernel,
        out_type=jax.ShapeDtypeStruct((NROWS * CROP,), jnp.float32),
        mesh=mesh,
        compiler_params=pltpu.CompilerParams(needs_layout_passes=False),
        scratch_types=[
            pltpu.VMEM((2 * GROUP,), jnp.int32),
            pltpu.VMEM((2 * GROUP,), jnp.int32),
            pltpu.VMEM((B * CROP,), jnp.int32),
            pltpu.VMEM((B * CROP,), jnp.float32),
            pltpu.VMEM((ROWS_PER_SUB,), jnp.float32),
            pltpu.VMEM((16,), jnp.int32),
            pltpu.VMEM((2 * GROUP, WIN), jnp.float32),
            pltpu.VMEM((2 * GROUP, WIN), jnp.float32),
            pltpu.VMEM((GROUP * CROP,), jnp.float32),
            pltpu.SemaphoreType.DMA,
            pltpu.SemaphoreType.DMA,
        ],
    )
    def k(img_hbm, idx_hbm, xloc_hbm, wx1_hbm, wy_hbm, c0_hbm, out_hbm,
          idx_a, idx_b, xloc_v, wx1_v, wy_v, c0_v, buf_a, buf_b, outb_v,
          sem_a, sem_b):
        wid = lax.axis_index("s") * 2 + lax.axis_index("c")
        rbase = wid * ROWS_PER_SUB
        b_s = wid // (NSUB // B)
        toff = b_s * CROP
        pltpu.sync_copy(xloc_hbm, xloc_v)
        pltpu.sync_copy(wx1_hbm, wx1_v)
        pltpu.sync_copy(wy_hbm.at[pl.ds(rbase, ROWS_PER_SUB)], wy_v)
        pltpu.sync_copy(c0_hbm, c0_v)
        c0s = c0_v[...]
        c0 = jnp.sum(jnp.where(lax.iota(jnp.int32, 16) == b_s, c0s, 0)) * 128
        img_w = img_hbm.at[:, pl.ds(c0, WIN)]

        def fetch(g, idx_v, buf_v, sem):
            pltpu.sync_copy(idx_hbm.at[pl.ds((rbase + g * GROUP) * 2,
                                             2 * GROUP)], idx_v)
            return pltpu.async_copy(img_w.at[idx_v], buf_v, sem)

        def compute(g, buf_v):
            for t in range(GROUP):
                ridx = jnp.full((16,), g * GROUP + t, dtype=jnp.int32)
                wyv = plsc.load_gather(wy_v, [ridx])
                ra = jnp.full((16,), 2 * t, dtype=jnp.int32)
                rb = jnp.full((16,), 2 * t + 1, dtype=jnp.int32)

                def chunk(xi, w1):
                    xi1 = xi + 1
                    a0 = plsc.load_gather(buf_v, [ra, xi])
                    a1 = plsc.load_gather(buf_v, [ra, xi1])
                    b0 = plsc.load_gather(buf_v, [rb, xi])
                    b1 = plsc.load_gather(buf_v, [rb, xi1])
                    top = a0 + w1 * (a1 - a0)
                    bot = b0 + w1 * (b1 - b0)
                    return top + wyv * (bot - top)

                def chunk4(k4, carry2):
                    for u in range(4):
                        off = toff + k4 * 64 + u * 16
                        xi = xloc_v[pl.ds(off, 16)]
                        w1 = wx1_v[pl.ds(off, 16)]
                        outb_v[pl.ds(t * CROP + k4 * 64 + u * 16, 16)] = (
                            chunk(xi, w1))
                    return carry2

                lax.fori_loop(0, CROP // 64, chunk4, 0)

            pltpu.sync_copy(outb_v,
                            out_hbm.at[pl.ds((rbase + g * GROUP) * CROP,
                                             GROUP * CROP)])

        fetch(0, idx_a, buf_a, sem_a)

        def pair_body(i, carry):
            g0 = 2 * i
            fetch(g0 + 1, idx_b, buf_b, sem_b)
            pltpu.make_async_copy(img_w.at[idx_a], buf_a, sem_a).wait()
            compute(g0, buf_a)

            @pl.when(g0 + 2 < NGROUPS)
            def _():
                fetch(g0 + 2, idx_a, buf_a, sem_a)

            pltpu.make_async_copy(img_w.at[idx_b], buf_b, sem_b).wait()
            compute(g0 + 1, buf_b)
            return carry

        lax.fori_loop(0, NGROUPS // 2, pair_body, 0)

    return k(img_rows, idx_all, xloc, wx1, wy, c0tab)


def kernel(full_imgs, center, bbox_size):
    crop = CROP
    zeros = jnp.zeros((B,), dtype=full_imgs.dtype)
    ones = jnp.ones((B,), dtype=full_imgs.dtype)
    s = bbox_size
    cx = center[:, 0]
    cy = center[:, 1]
    transforms = jnp.stack([
        jnp.stack([s, zeros, cx - s * 0.5], axis=1),
        jnp.stack([zeros, s, cy - s * 0.5], axis=1),
        jnp.stack([zeros, zeros, ones], axis=1)], axis=1)
    a = 2.0 * (crop - 1) / s
    hd_to_crop = jnp.stack([
        jnp.stack([a, zeros, -(cx - s * 0.5) * a - 1.0], axis=1),
        jnp.stack([zeros, a, -(cy - s * 0.5) * a - 1.0], axis=1),
        jnp.stack([zeros, zeros, ones], axis=1)], axis=1)

    sx = 2.0 / (W - 1) * ones
    sy = 2.0 / (H - 1) * ones
    size_bbox_sizer = jnp.stack([
        jnp.stack([sx, zeros, -ones], axis=1),
        jnp.stack([zeros, sy, -ones], axis=1),
        jnp.stack([zeros, zeros, ones], axis=1)], axis=1)
    full_transform = jnp.einsum('bij,bjk->bik', size_bbox_sizer, transforms)
    x1d = jnp.arange(crop, dtype=jnp.float32) / (crop - 1)
    gy, gx = jnp.meshgrid(x1d, x1d, indexing='ij')
    points = jnp.stack([gy.reshape(-1), gx.reshape(-1)], axis=1)
    batch_grid = jnp.broadcast_to(points[None], (B, crop * crop, 2))
    sg = (jnp.einsum('bij,bnj->bni', full_transform[:, :2, :2], batch_grid)
          + full_transform[:, :2, 2][:, None, :])
    sampling_grid = jnp.swapaxes(sg.reshape(B, crop, crop, 2), 1, 2)

    sg_x = sampling_grid[:, 0, :, 0]
    sg_y = sampling_grid[:, :, 0, 1]
    ix = (sg_x + 1.0) * (W - 1) / 2.0
    iy = (sg_y + 1.0) * (H - 1) / 2.0
    x0f = jnp.floor(ix)
    y0f = jnp.floor(iy)
    x0 = x0f.astype(jnp.int32)
    y0 = y0f.astype(jnp.int32)
    wx1 = ix - x0f
    wy1 = iy - y0f

    bc = (jnp.arange(B, dtype=jnp.int32)[:, None] * C
          + jnp.arange(C, dtype=jnp.int32)[None, :])
    rowid = bc[:, :, None] * H + y0[:, None, :]
    idx_all = jnp.stack([rowid, rowid + 1], axis=-1).reshape(-1)

    wy_flat = jnp.broadcast_to(wy1[:, None, :], (B, C, crop)).reshape(-1)

    c0blk = jnp.minimum(x0[:, 0] >> 7, (W - WIN) // 128)
    c0tab = jnp.pad(c0blk, (0, 16 - B))
    xloc = x0 - (c0blk << 7)[:, None]

    img_rows = full_imgs.reshape(B * C * H, W)
    out_flat = _sc_bilinear(img_rows, idx_all, xloc.reshape(-1),
                            wx1.reshape(-1), wy_flat, c0tab)
    out_images = out_flat.reshape(B, C, crop, crop)

    return (out_images, sampling_grid.reshape(B, -1, 2), transforms,
            hd_to_crop)


# speedup vs baseline: 5.1950x; 1.1532x over previous
import functools

import jax
import jax.numpy as jnp
from jax import lax
from jax.experimental import pallas as pl
from jax.experimental.pallas import tpu as pltpu
from jax.experimental.pallas import tpu_sc as plsc

CROP = 256
B, C, H, W = 4, 3, 2048, 2048
NROWS = B * C * CROP
NSUB = 32
ROWS_PER_SUB = NROWS // NSUB
GROUP = 8
NGROUPS = ROWS_PER_SUB // GROUP


WIN = 1024


def _sc_bilinear(img_rows, idx_all, xloc, wx1, wy, c0tab):
    mesh = plsc.VectorSubcoreMesh(core_axis_name="c", subcore_axis_name="s",
                                  num_cores=2, num_subcores=16)

    @functools.partial(
        pl.kernel,
        out_type=jax.ShapeDtypeStruct((NROWS * CROP,), jnp.float32),
        mesh=mesh,
        compiler_params=pltpu.CompilerParams(needs_layout_passes=False),
        scratch_types=[
            pltpu.VMEM((2 * GROUP,), jnp.int32),
            pltpu.VMEM((2 * GROUP,), jnp.int32),
            pltpu.VMEM((B * CROP,), jnp.int32),
            pltpu.VMEM((B * CROP,), jnp.float32),
            pltpu.VMEM((ROWS_PER_SUB,), jnp.float32),
            pltpu.VMEM((16,), jnp.int32),
            pltpu.VMEM((2 * GROUP, WIN), jnp.float32),
            pltpu.VMEM((2 * GROUP, WIN), jnp.float32),
            pltpu.VMEM((GROUP * CROP,), jnp.float32),
            pltpu.SemaphoreType.DMA,
            pltpu.SemaphoreType.DMA,
        ],
    )
    def k(img_hbm, idx_hbm, xloc_hbm, wx1_hbm, wy_hbm, c0_hbm, out_hbm,
          idx_a, idx_b, xloc_v, wx1_v, wy_v, c0_v, buf_a, buf_b, outb_v,
          sem_a, sem_b):
        wid = lax.axis_index("s") * 2 + lax.axis_index("c")
        rbase = wid * ROWS_PER_SUB
        b_s = wid // (NSUB // B)
        toff = b_s * CROP
        pltpu.sync_copy(xloc_hbm, xloc_v)
        pltpu.sync_copy(wx1_hbm, wx1_v)
        pltpu.sync_copy(wy_hbm.at[pl.ds(rbase, ROWS_PER_SUB)], wy_v)
        pltpu.sync_copy(c0_hbm, c0_v)
        c0s = c0_v[...]
        c0 = jnp.sum(jnp.where(lax.iota(jnp.int32, 16) == b_s, c0s, 0)) * 128
        img_w = img_hbm.at[:, pl.ds(c0, WIN)]

        def fetch(g, idx_v, buf_v, sem):
            pltpu.sync_copy(idx_hbm.at[pl.ds((rbase + g * GROUP) * 2,
                                             2 * GROUP)], idx_v)
            return pltpu.async_copy(img_w.at[idx_v], buf_v, sem)

        def compute(g, buf_v):
            wyvs = []
            for t in range(GROUP):
                ridx = jnp.full((16,), g * GROUP + t, dtype=jnp.int32)
                wyvs.append(plsc.load_gather(wy_v, [ridx]))

            def chunk4(k4, carry2):
                for u in range(4):
                    off = k4 * 64 + u * 16
                    xi = xloc_v[pl.ds(toff + off, 16)]
                    w1 = wx1_v[pl.ds(toff + off, 16)]
                    xi1 = xi + 1
                    for t in range(GROUP):
                        ra = jnp.full((16,), 2 * t, dtype=jnp.int32)
                        rb = jnp.full((16,), 2 * t + 1, dtype=jnp.int32)
                        a0 = plsc.load_gather(buf_v, [ra, xi])
                        a1 = plsc.load_gather(buf_v, [ra, xi1])
                        b0 = plsc.load_gather(buf_v, [rb, xi])
                        b1 = plsc.load_gather(buf_v, [rb, xi1])
                        top = a0 + w1 * (a1 - a0)
                        bot = b0 + w1 * (b1 - b0)
                        outb_v[pl.ds(t * CROP + off, 16)] = (
                            top + wyvs[t] * (bot - top))
                return carry2

            lax.fori_loop(0, CROP // 64, chunk4, 0)

            pltpu.sync_copy(outb_v,
                            out_hbm.at[pl.ds((rbase + g * GROUP) * CROP,
                                             GROUP * CROP)])

        fetch(0, idx_a, buf_a, sem_a)

        def pair_body(i, carry):
            g0 = 2 * i
            fetch(g0 + 1, idx_b, buf_b, sem_b)
            pltpu.make_async_copy(img_w.at[idx_a], buf_a, sem_a).wait()
            compute(g0, buf_a)

            @pl.when(g0 + 2 < NGROUPS)
            def _():
                fetch(g0 + 2, idx_a, buf_a, sem_a)

            pltpu.make_async_copy(img_w.at[idx_b], buf_b, sem_b).wait()
            compute(g0 + 1, buf_b)
            return carry

        lax.fori_loop(0, NGROUPS // 2, pair_body, 0)

    return k(img_rows, idx_all, xloc, wx1, wy, c0tab)


def kernel(full_imgs, center, bbox_size):
    crop = CROP
    zeros = jnp.zeros((B,), dtype=full_imgs.dtype)
    ones = jnp.ones((B,), dtype=full_imgs.dtype)
    s = bbox_size
    cx = center[:, 0]
    cy = center[:, 1]
    transforms = jnp.stack([
        jnp.stack([s, zeros, cx - s * 0.5], axis=1),
        jnp.stack([zeros, s, cy - s * 0.5], axis=1),
        jnp.stack([zeros, zeros, ones], axis=1)], axis=1)
    a = 2.0 * (crop - 1) / s
    hd_to_crop = jnp.stack([
        jnp.stack([a, zeros, -(cx - s * 0.5) * a - 1.0], axis=1),
        jnp.stack([zeros, a, -(cy - s * 0.5) * a - 1.0], axis=1),
        jnp.stack([zeros, zeros, ones], axis=1)], axis=1)

    sx = 2.0 / (W - 1) * ones
    sy = 2.0 / (H - 1) * ones
    size_bbox_sizer = jnp.stack([
        jnp.stack([sx, zeros, -ones], axis=1),
        jnp.stack([zeros, sy, -ones], axis=1),
        jnp.stack([zeros, zeros, ones], axis=1)], axis=1)
    full_transform = jnp.einsum('bij,bjk->bik', size_bbox_sizer, transforms)
    x1d = jnp.arange(crop, dtype=jnp.float32) / (crop - 1)
    gy, gx = jnp.meshgrid(x1d, x1d, indexing='ij')
    points = jnp.stack([gy.reshape(-1), gx.reshape(-1)], axis=1)
    batch_grid = jnp.broadcast_to(points[None], (B, crop * crop, 2))
    sg = (jnp.einsum('bij,bnj->bni', full_transform[:, :2, :2], batch_grid)
          + full_transform[:, :2, 2][:, None, :])
    sampling_grid = jnp.swapaxes(sg.reshape(B, crop, crop, 2), 1, 2)

    sg_x = sampling_grid[:, 0, :, 0]
    sg_y = sampling_grid[:, :, 0, 1]
    ix = (sg_x + 1.0) * (W - 1) / 2.0
    iy = (sg_y + 1.0) * (H - 1) / 2.0
    x0f = jnp.floor(ix)
    y0f = jnp.floor(iy)
    x0 = x0f.astype(jnp.int32)
    y0 = y0f.astype(jnp.int32)
    wx1 = ix - x0f
    wy1 = iy - y0f

    bc = (jnp.arange(B, dtype=jnp.int32)[:, None] * C
          + jnp.arange(C, dtype=jnp.int32)[None, :])
    rowid = bc[:, :, None] * H + y0[:, None, :]
    idx_all = jnp.stack([rowid, rowid + 1], axis=-1).reshape(-1)

    wy_flat = jnp.broadcast_to(wy1[:, None, :], (B, C, crop)).reshape(-1)

    c0blk = jnp.minimum(x0[:, 0] >> 7, (W - WIN) // 128)
    c0tab = jnp.pad(c0blk, (0, 16 - B))
    xloc = x0 - (c0blk << 7)[:, None]

    img_rows = full_imgs.reshape(B * C * H, W)
    out_flat = _sc_bilinear(img_rows, idx_all, xloc.reshape(-1),
                            wx1.reshape(-1), wy_flat, c0tab)
    out_images = out_flat.reshape(B, C, crop, crop)

    return (out_images, sampling_grid.reshape(B, -1, 2), transforms,
            hd_to_crop)
